# baseline (speedup 1.0000x reference)
import jax
import jax.numpy as jnp
from jax import lax
from jax.experimental import pallas as pl
from jax.experimental.pallas import tpu as pltpu
from jax.experimental.pallas import tpu_sc as plsc

_NC, _NS = 2, 16
_NW = _NC * _NS


def _bcast(i16, scalar):
    return i16 * 0 + scalar


def _gb_body(tp, idx, out, idxf, ir0, ir1, pb0, pb1, ob0, ob1,
             gs0, gs1, ws0, ws1):
    wid = lax.axis_index("s") * _NC + lax.axis_index("c")
    i16 = lax.iota(jnp.int32, 16)
    irs, pbs, obs = (ir0, ir1), (pb0, pb1), (ob0, ob1)
    gsems, wsems = (gs0, gs1), (ws0, ws1)

    pltpu.sync_copy(idx.at[pl.ds(wid * 6400, 6400)], idxf)
    pat50 = i16 * 50

    def stage_idx(h, b):
        for g in range(8):
            addr = pat50 + (g * 16 * 50 + h)
            irs[b][pl.ds(g * 16, 16)] = plsc.load_gather(idxf, [addr])
        pltpu.async_copy(tp.at[irs[b]], pbs[b], gsems[b])

    def wait_gather(b):
        pltpu.make_async_copy(tp.at[pl.ds(0, 128), :], pbs[b], gsems[b]).wait()

    def shuffle_out(b):
        @plsc.parallel_loop(0, 64, unroll=8)
        def row(d):
            col = _bcast(i16, d)
            for g in range(8):
                x = plsc.load_gather(pbs[b], [i16 + g * 16, col])
                obs[b][d, pl.ds(g * 16, 16)] = x

    def start_write(h, b):
        pltpu.async_copy(obs[b], out.at[h, :, pl.ds(wid * 128, 128)], wsems[b])

    def wait_write(b):
        pltpu.make_async_copy(tp.at[pl.ds(0, 64), :], obs[b], wsems[b]).wait()

    stage_idx(0, 0)
    stage_idx(1, 1)
    for b in range(2):
        wait_gather(b)
        shuffle_out(b)
        start_write(b, b)
        stage_idx(b + 2, b)

    def step(h2, carry):
        for b in range(2):
            h = 2 * h2 + b
            wait_gather(b)
            wait_write(b)
            shuffle_out(b)
            start_write(h, b)
            stage_idx(h + 2, b)
        return carry

    lax.fori_loop(1, 24, step, 0)

    for b in range(2):
        wait_gather(b)
        wait_write(b)
        shuffle_out(b)
        start_write(48 + b, b)
    for b in range(2):
        wait_write(b)


def kernel(x, embed_weight):
    B, H = x.shape
    V, D = embed_weight.shape
    mesh = plsc.VectorSubcoreMesh(core_axis_name="c", subcore_axis_name="s")
    params = pltpu.CompilerParams(
        use_tc_tiling_on_sc=True, needs_layout_passes=False
    )

    tp = jnp.zeros((V, 128), jnp.float32).at[:, :D].set(embed_weight)
    idx = x.reshape(B * H).astype(jnp.int32)

    gather_fn = pl.kernel(
        _gb_body,
        out_type=jax.ShapeDtypeStruct((H, D, B), jnp.float32),
        mesh=mesh,
        compiler_params=params,
        scratch_types=[
            pltpu.VMEM((6400,), jnp.int32),
            pltpu.VMEM((128,), jnp.int32),
            pltpu.VMEM((128,), jnp.int32),
            pltpu.VMEM((128, 128), jnp.float32),
            pltpu.VMEM((128, 128), jnp.float32),
            pltpu.VMEM((64, 128), jnp.float32),
            pltpu.VMEM((64, 128), jnp.float32),
            pltpu.SemaphoreType.DMA,
            pltpu.SemaphoreType.DMA,
            pltpu.SemaphoreType.DMA,
            pltpu.SemaphoreType.DMA,
        ],
    )
    outb = gather_fn(tp, idx)
    return jnp.transpose(outb, (2, 0, 1))
